# speedup vs baseline: 1.4814x; 1.4814x over previous
import jax
import jax.numpy as jnp
from jax.experimental import pallas as pl
from jax.experimental.pallas import tpu as pltpu

_H0, _W0 = 56, 120
_N_CAM, _M_HEADS, _HG, _WG = 6, 4, 25, 25
_K = _N_CAM * _H0 * _W0


def _body(att_ref, y_ref, ood_ref, out_ref):
    j = pl.program_id(1)

    a = att_ref[0, :, 0]
    s = a[0] + a[1] + a[2] + a[3]

    maxv = jnp.max(s, axis=-1, keepdims=True)
    ki = jax.lax.broadcasted_iota(jnp.int32, (_WG, _K), 1)
    cand = jnp.where(s == maxv, ki, _K)
    idx = jnp.min(cand, axis=-1, keepdims=True)

    ood_row = ood_ref[0]
    val = jnp.sum(jnp.where(ki == idx, ood_row, 0.0), axis=-1)

    p = val[None, :]
    logp = jnp.maximum(jnp.log(p), -100.0)
    log1mp = jnp.maximum(jnp.log1p(-p), -100.0)

    col_cell = jax.lax.broadcasted_iota(jnp.int32, (_WG, 8 * _WG), 1) // 8
    row_id = jax.lax.broadcasted_iota(jnp.int32, (_WG, 8 * _WG), 0)
    g = (col_cell == row_id).astype(jnp.float32)
    logp_row = jnp.dot(logp, g, preferred_element_type=jnp.float32)
    log1mp_row = jnp.dot(log1mp, g, preferred_element_type=jnp.float32)

    t = y_ref[0, 0]
    tc = jnp.sum(t, axis=0, keepdims=True)
    partial = -(jnp.sum(tc * logp_row) + jnp.sum((8.0 - tc) * log1mp_row))

    @pl.when(j == 0)
    def _():
        out_ref[...] = jnp.zeros_like(out_ref)

    out_ref[...] += partial

def kernel(alpha, y, ood, ood_cam, att0, att1):
    B = y.shape[0]
    att = att0.reshape(B, _M_HEADS, _HG, _WG, _K)
    ood_flat = ood_cam.reshape(B, 1, _K)

    out = pl.pallas_call(
        _body,
        grid=(B, _HG),
        in_specs=[
            pl.BlockSpec((1, _M_HEADS, 1, _WG, _K),
                         lambda b, j: (b, 0, j, 0, 0)),
            pl.BlockSpec((1, 1, 8, 8 * _WG), lambda b, j: (b, 0, j, 0)),
            pl.BlockSpec((1, 1, _K), lambda b, j: (b, 0, 0)),
        ],
        out_specs=pl.BlockSpec((1, 1, 1), lambda b, j: (b, 0, 0)),
        out_shape=jax.ShapeDtypeStruct((B, 1, 1), jnp.float32),
        compiler_params=pltpu.CompilerParams(
            dimension_semantics=("parallel", "arbitrary"),
            vmem_limit_bytes=56 * 1024 * 1024,
        ),
    )(att, y, ood_flat)

    return out.sum() / (B * 8 * _HG * 8 * _WG)


# speedup vs baseline: 3.1884x; 2.1522x over previous
import jax
import jax.numpy as jnp
from jax.experimental import pallas as pl
from jax.experimental.pallas import tpu as pltpu

_H0, _W0 = 56, 120
_N_CAM, _M_HEADS, _HG, _WG = 6, 4, 25, 25
_P = _HG * _WG
_K = _N_CAM * _H0 * _W0
_KB = 1920
_NK = _K // _KB


def _argmax_body(att_ref, ood_ref, out_ref, rmax_ref, rval_ref):
    k = pl.program_id(1)

    @pl.when(k == 0)
    def _():
        rmax_ref[...] = jnp.full_like(rmax_ref, -jnp.inf)
        rval_ref[...] = jnp.zeros_like(rval_ref)

    a = att_ref[...]
    s = a[0] + a[1] + a[2] + a[3]

    bmax = jnp.max(s, axis=-1, keepdims=True)
    ki = jax.lax.broadcasted_iota(jnp.int32, (_P, _KB), 1)
    cand = jnp.where(s == bmax, ki, _KB)
    bidx = jnp.min(cand, axis=-1, keepdims=True)
    ood_blk = ood_ref[0]
    bval = jnp.sum(jnp.where(ki == bidx, ood_blk, 0.0),
                   axis=-1, keepdims=True)

    upd = bmax > rmax_ref[...]
    rmax_ref[...] = jnp.where(upd, bmax, rmax_ref[...])
    rval_ref[...] = jnp.where(upd, bval, rval_ref[...])

    @pl.when(k == _NK - 1)
    def _():
        out_ref[0] = rval_ref[...]


def _bce_body(mask_ref, y_ref, out_ref):
    m = mask_ref[0]
    t = y_ref[0, 0]
    logp = jnp.maximum(jnp.log(m), -100.0)
    log1mp = jnp.maximum(jnp.log1p(-m), -100.0)

    cell = jax.lax.broadcasted_iota(jnp.int32, (_WG, 8 * _WG), 1) // 8
    row = jax.lax.broadcasted_iota(jnp.int32, (_WG, 8 * _WG), 0)
    g = (cell == row).astype(jnp.float32)
    logp_w = jnp.dot(logp, g, preferred_element_type=jnp.float32)
    log1mp_w = jnp.dot(log1mp, g, preferred_element_type=jnp.float32)
    tc = jnp.dot(g, t, preferred_element_type=jnp.float32)
    out_ref[...] = -(jnp.sum(tc * logp_w)
                     + jnp.sum((8.0 - tc) * log1mp_w))[None, None, None]


def kernel(alpha, y, ood, ood_cam, att0, att1):
    B = y.shape[0]
    ood_flat = ood_cam.reshape(B, 1, _K)

    rval = pl.pallas_call(
        _argmax_body,
        grid=(B, _NK),
        in_specs=[
            pl.BlockSpec((_M_HEADS, _P, _KB), lambda b, k: (b, 0, k)),
            pl.BlockSpec((1, 1, _KB), lambda b, k: (b, 0, k)),
        ],
        out_specs=pl.BlockSpec((1, _P, 1), lambda b, k: (b, 0, 0)),
        out_shape=jax.ShapeDtypeStruct((B, _P, 1), jnp.float32),
        scratch_shapes=[
            pltpu.VMEM((_P, 1), jnp.float32),
            pltpu.VMEM((_P, 1), jnp.float32),
        ],
        compiler_params=pltpu.CompilerParams(
            dimension_semantics=("parallel", "arbitrary"),
            vmem_limit_bytes=56 * 1024 * 1024,
        ),
    )(att0, ood_flat)

    mask = rval.reshape(B, _HG, _WG)

    out = pl.pallas_call(
        _bce_body,
        grid=(B,),
        in_specs=[
            pl.BlockSpec((1, _HG, _WG), lambda b: (b, 0, 0)),
            pl.BlockSpec((1, 1, 8 * _HG, 8 * _WG), lambda b: (b, 0, 0, 0)),
        ],
        out_specs=pl.BlockSpec((1, 1, 1), lambda b: (b, 0, 0)),
        out_shape=jax.ShapeDtypeStruct((B, 1, 1), jnp.float32),
        compiler_params=pltpu.CompilerParams(
            dimension_semantics=("parallel",),
        ),
    )(mask, y)

    return out.sum() / (B * 8 * _HG * 8 * _WG)
